# baseline (speedup 1.0000x reference)
import functools
import jax
import jax.numpy as jnp
from jax import lax
from jax.experimental import pallas as pl
from jax.experimental.pallas import tpu as pltpu
from jax.experimental.pallas import tpu_sc as plsc

NSJ = 50000
EJ = 800000
NP = 50176
EP = 802816
NEG = -3.0e38


@functools.lru_cache(maxsize=None)
def _sc_gather(n_rows, d, b_per_w, ch):
    mesh = plsc.VectorSubcoreMesh(core_axis_name="c", subcore_axis_name="s")
    n_it = b_per_w // ch

    @functools.partial(
        pl.kernel, mesh=mesh,
        out_type=jax.ShapeDtypeStruct((n_rows * 0 + b_per_w * 32, d), jnp.float32),
        scratch_types=[
            pltpu.VMEM((ch,), jnp.int32),
            pltpu.VMEM((ch, d), jnp.float32),
            pltpu.SemaphoreType.DMA,
        ],
    )
    def k(table_hbm, idx_hbm, out_hbm, idx_v, rows_v, sem):
        wid = lax.axis_index("s") * 2 + lax.axis_index("c")
        base = wid * b_per_w

        def body(i, carry):
            off = base + i * ch
            pltpu.sync_copy(idx_hbm.at[pl.ds(off, ch)], idx_v)
            pltpu.async_copy(table_hbm.at[idx_v], rows_v, sem).wait()
            pltpu.sync_copy(rows_v, out_hbm.at[pl.ds(off, ch)])
            return carry

        lax.fori_loop(0, n_it, body, 0)

    return k


def _gather_rows(table, idx, b_per_w, ch):
    return _sc_gather(table.shape[0], table.shape[1], b_per_w, ch)(table, idx)


def _scan_body(is_max, g_ref, d_ref, out_ref, cval, cdst):
    b = pl.program_id(0)
    ident = NEG if is_max else 0.0

    @pl.when(b == 0)
    def _():
        cval[...] = jnp.full_like(cval[...], ident)
        cdst[...] = jnp.full_like(cdst[...], -1.0)

    v = g_ref[...]
    dst = d_ref[...]
    B = v.shape[0]

    d_prev = jnp.concatenate([cdst[...], dst[:-1]], axis=0)
    f = jnp.minimum(jnp.abs(dst - d_prev), 1.0)

    row0 = jnp.concatenate(
        [jnp.ones((1, 1), jnp.float32), jnp.zeros((B - 1, 1), jnp.float32)], axis=0)
    cont0 = row0 * (1.0 - f)
    cv = jnp.broadcast_to(cval[...], v.shape)
    if is_max:
        v = jnp.maximum(v, cont0 * cv + (1.0 - cont0) * NEG)
    else:
        v = v + cont0 * cv

    s = 1
    while s < B:
        pad_v = jnp.full((s, v.shape[1]), ident, jnp.float32)
        pad_f = jnp.ones((s, 1), jnp.float32)
        v_s = jnp.concatenate([pad_v, v[:-s]], axis=0)
        f_s = jnp.concatenate([pad_f, f[:-s]], axis=0)
        comb = jnp.maximum(v, v_s) if is_max else v + v_s
        v = f * v + (1.0 - f) * comb
        f = jnp.maximum(f, f_s)
        s *= 2

    out_ref[...] = v
    cval[...] = v[-1:, :]
    cdst[...] = dst[-1:, :]


@functools.lru_cache(maxsize=None)
def _seg_scan(d, is_max):
    B = 1024
    grid = EP // B
    return pl.pallas_call(
        functools.partial(_scan_body, is_max),
        grid=(grid,),
        in_specs=[
            pl.BlockSpec((B, d), lambda b: (b, 0)),
            pl.BlockSpec((B, 1), lambda b: (b, 0)),
        ],
        out_specs=pl.BlockSpec((B, d), lambda b: (b, 0)),
        out_shape=jax.ShapeDtypeStruct((EP, d), jnp.float32),
        scratch_shapes=[
            pltpu.VMEM((1, d), jnp.float32),
            pltpu.VMEM((1, 1), jnp.float32),
        ],
    )


def _proj_body(x_ref, w_ref, o_ref):
    o_ref[...] = jax.lax.dot_general(
        x_ref[...], w_ref[...], (((1,), (0,)), ((), ())),
        preferred_element_type=jnp.float32)


@functools.lru_cache(maxsize=None)
def _proj(di, do):
    R = 1568
    return pl.pallas_call(
        _proj_body,
        grid=(NP // R,),
        in_specs=[
            pl.BlockSpec((R, di), lambda b: (b, 0)),
            pl.BlockSpec((di, do), lambda b: (0, 0)),
        ],
        out_specs=pl.BlockSpec((R, do), lambda b: (b, 0)),
        out_shape=jax.ShapeDtypeStruct((NP, do), jnp.float32),
    )


def _dense_body(mode, xs_ref, xb_ref, z0_ref, z1_ref, z2_ref, m3_ref,
                c1_ref, c2_ref, c3_ref, wr0_ref, wr1_ref, wr2_ref,
                wrs_ref, wrb_ref, bs_ref, bb_ref, hps_ref, hpb_ref,
                wres_s_ref, wres_b_ref, wlin_ref, blin_ref,
                os_ref, ob_ref):
    dot = lambda a, b: jax.lax.dot_general(
        a, b, (((1,), (0,)), ((), ())), preferred_element_type=jnp.float32)
    c1 = c1_ref[...]
    c2 = c2_ref[...]
    c3 = c3_ref[...]
    z1 = z1_ref[...] / jnp.maximum(c1, 1.0)
    z2 = z2_ref[...] / jnp.maximum(c2, 1.0)
    m3 = jnp.minimum(c3, 1.0) * m3_ref[...]

    ob = jax.nn.relu(dot(z2, wr2_ref[...]) + m3 +
                     dot(xb_ref[...], wrb_ref[...]) + bb_ref[...])
    if mode == "res_mm":
        ob = ob + dot(hpb_ref[...], wres_b_ref[...])
    elif mode == "res_id":
        ob = ob + hpb_ref[...]
    if mode == "final":
        ob = ob + hpb_ref[...]
        ob = dot(ob, wlin_ref[...]) + blin_ref[...]
    ob_ref[...] = ob

    if mode == "final":
        os_ref[...] = jnp.zeros_like(os_ref[...])
    else:
        os = jax.nn.relu(dot(z0_ref[...], wr0_ref[...]) + dot(z1, wr1_ref[...]) +
                         dot(xs_ref[...], wrs_ref[...]) + bs_ref[...])
        if mode == "res_mm":
            os = os + dot(hps_ref[...], wres_s_ref[...])
        elif mode == "res_id":
            os = os + hps_ref[...]
        os_ref[...] = os


@functools.lru_cache(maxsize=None)
def _dense(di, do, dp, dof, mode):
    R = 1568
    row = lambda d: pl.BlockSpec((R, d), lambda b: (b, 0))
    full = lambda a, c: pl.BlockSpec((a, c), lambda b: (0, 0))
    return pl.pallas_call(
        functools.partial(_dense_body, mode),
        grid=(NP // R,),
        in_specs=[
            row(di), row(di), row(di), row(di), row(di), row(do),
            row(1), row(1), row(1),
            full(di, do), full(di, do), full(di, do),
            full(di, do), full(di, do),
            full(1, do), full(1, do),
            row(dp), row(dp),
            full(dp, do), full(dp, do),
            full(do, dof), full(1, dof),
        ],
        out_specs=[row(do), row(dof)],
        out_shape=[jax.ShapeDtypeStruct((NP, do), jnp.float32),
                   jax.ShapeDtypeStruct((NP, dof), jnp.float32)],
    )


def _prep_edges(e):
    src = e[0].astype(jnp.int32)
    dst = e[1].astype(jnp.int32)
    perm = jnp.argsort(dst)
    src_s = src[perm]
    dst_s = dst[perm]
    pad_src = (jnp.arange(EP - EJ, dtype=jnp.int32) * 7919) % NSJ
    src_p = jnp.concatenate([src_s, pad_src])
    dst_p = jnp.concatenate([dst_s, jnp.full((EP - EJ,), NP + 5, jnp.int32)])
    off = jnp.searchsorted(dst_s, jnp.arange(NP + 1, dtype=jnp.int32)).astype(jnp.int32)
    last = jnp.maximum(off[1:] - 1, 0)
    cnt = (off[1:] - off[:-1]).astype(jnp.float32)[:, None]
    dst_f = dst_p.astype(jnp.float32)[:, None]
    return src_p, dst_f, last, cnt


def _segment_reduce(x_pad, src_p, dst_f, last, is_max):
    g = _gather_rows(x_pad, src_p, 25088, 512)
    s = _seg_scan(x_pad.shape[1], is_max)(g, dst_f)
    return _gather_rows(s, last, 1568, 784)


def _pad_rows(x, rows):
    return jnp.pad(x, ((0, rows - x.shape[0]), (0, 0)))


def _padw(w):
    return jnp.pad(w, ((0, 128 - w.shape[0]), (0, 128 - w.shape[1])))


def kernel(x_stroke, x_brep, edge_index_temp_previous, edge_index_intersects,
           edge_index_represented_by, edge_index_brepcoplanar,
           Wrel0, Wroot0, b0, Wrel1, Wroot1, b1, Wrel2, Wroot2, b2,
           Wrel3, Wroot3, b3, Wrel4, Wroot4, b4, Wres1, Wlin, blin):
    e0 = _prep_edges(edge_index_temp_previous)
    e1 = _prep_edges(edge_index_intersects)
    e2 = _prep_edges(edge_index_represented_by)
    e3 = _prep_edges(edge_index_brepcoplanar)

    xs = _pad_rows(jnp.pad(x_stroke, ((0, 0), (0, 122))), NP)
    xb = _pad_rows(jnp.pad(x_brep, ((0, 0), (0, 122))), NP)

    pw = lambda W: [_padw(W[i]) for i in range(4)]
    pb = lambda b: jnp.pad(b, ((0, 0), (0, 128 - b.shape[1])))
    zero = jnp.zeros((128, 128), jnp.float32)
    zrow = jnp.zeros((NP, 128), jnp.float32)

    layers = [
        (pw(Wrel0), pw(Wroot0), pb(b0), "plain"),
        (pw(Wrel1), pw(Wroot1), pb(b1), "res_mm"),
        (pw(Wrel2), pw(Wroot2), pb(b2), "res_id"),
        (pw(Wrel3), pw(Wroot3), pb(b3), "res_id"),
        (pw(Wrel4), pw(Wroot4), pb(b4), "final"),
    ]

    hs, hb = xs, xb
    for (Wr, Wq, bias, mode) in layers:
        z0 = _segment_reduce(hs, e0[0], e0[1], e0[2], False)
        z1 = _segment_reduce(hs, e1[0], e1[1], e1[2], False)
        z2 = _segment_reduce(hs, e2[0], e2[1], e2[2], False)
        y3 = _proj(128, 128)(hb, Wr[3])
        m3 = _segment_reduce(y3, e3[0], e3[1], e3[2], True)

        wrs = Wq[0] + Wq[1]
        wrb = Wq[2] + Wq[3]
        bs = (bias[0] + bias[1])[None, :]
        bb = (bias[2] + bias[3])[None, :]
        if mode == "res_mm":
            wres_s = _padw(Wres1[0])
            wres_b = _padw(Wres1[1])
        else:
            wres_s = zero
            wres_b = zero
        wlin = Wlin if mode == "final" else zero
        blin_ = blin[None, :] if mode == "final" else jnp.zeros((1, 128), jnp.float32)
        hps = zrow if mode == "plain" else hs
        hpb = zrow if mode == "plain" else hb

        os_, ob_ = _dense(128, 128, 128, 128, mode)(
            hs, hb, z0, z1, z2, m3, e1[3], e2[3], e3[3],
            Wr[0], Wr[1], Wr[2], wrs, wrb, bs, bb, hps, hpb,
            wres_s, wres_b, wlin, blin_)
        hs, hb = os_, ob_

    return hb[:NSJ]


# speedup vs baseline: 1.0002x; 1.0002x over previous
import functools
import jax
import jax.numpy as jnp
from jax import lax
from jax.experimental import pallas as pl
from jax.experimental.pallas import tpu as pltpu
from jax.experimental.pallas import tpu_sc as plsc

NSJ = 50000
EJ = 800000
NP = 50176
EP = 802816
NEG = -3.0e38


@functools.lru_cache(maxsize=None)
def _sc_gather(n_rows, d, b_per_w, ch):
    mesh = plsc.VectorSubcoreMesh(core_axis_name="c", subcore_axis_name="s")
    n_pair = b_per_w // (2 * ch)

    @functools.partial(
        pl.kernel, mesh=mesh,
        out_type=jax.ShapeDtypeStruct((n_rows * 0 + b_per_w * 32, d), jnp.float32),
        scratch_types=[
            pltpu.VMEM((ch,), jnp.int32),
            pltpu.VMEM((ch,), jnp.int32),
            pltpu.VMEM((ch, d), jnp.float32),
            pltpu.VMEM((ch, d), jnp.float32),
            pltpu.SemaphoreType.DMA,
            pltpu.SemaphoreType.DMA,
        ],
    )
    def k(table_hbm, idx_hbm, out_hbm, idx0_v, idx1_v, rows0_v, rows1_v, sem0, sem1):
        wid = lax.axis_index("s") * 2 + lax.axis_index("c")
        base = wid * b_per_w

        def body(p, carry):
            off = base + p * 2 * ch
            pltpu.sync_copy(idx_hbm.at[pl.ds(off, ch)], idx0_v)
            pltpu.sync_copy(idx_hbm.at[pl.ds(off + ch, ch)], idx1_v)
            c0 = pltpu.async_copy(table_hbm.at[idx0_v], rows0_v, sem0)
            c1 = pltpu.async_copy(table_hbm.at[idx1_v], rows1_v, sem1)
            c0.wait()
            pltpu.sync_copy(rows0_v, out_hbm.at[pl.ds(off, ch)])
            c1.wait()
            pltpu.sync_copy(rows1_v, out_hbm.at[pl.ds(off + ch, ch)])
            return carry

        lax.fori_loop(0, n_pair, body, 0)

    return k


def _gather_rows(table, idx, b_per_w, ch):
    return _sc_gather(table.shape[0], table.shape[1], b_per_w, ch)(table, idx)


def _scan_body(is_max, g_ref, d_ref, out_ref, cval, cdst):
    b = pl.program_id(0)
    ident = NEG if is_max else 0.0

    @pl.when(b == 0)
    def _():
        cval[...] = jnp.full_like(cval[...], ident)
        cdst[...] = jnp.full_like(cdst[...], -1.0)

    v = g_ref[...]
    dst = d_ref[...]
    B = v.shape[0]

    d_prev = jnp.concatenate([cdst[...], dst[:-1]], axis=0)
    f = jnp.minimum(jnp.abs(dst - d_prev), 1.0)

    row0 = jnp.concatenate(
        [jnp.ones((1, 1), jnp.float32), jnp.zeros((B - 1, 1), jnp.float32)], axis=0)
    cont0 = row0 * (1.0 - f)
    cv = jnp.broadcast_to(cval[...], v.shape)
    if is_max:
        v = jnp.maximum(v, cont0 * cv + (1.0 - cont0) * NEG)
    else:
        v = v + cont0 * cv

    s = 1
    while s < B:
        pad_v = jnp.full((s, v.shape[1]), ident, jnp.float32)
        pad_f = jnp.ones((s, 1), jnp.float32)
        v_s = jnp.concatenate([pad_v, v[:-s]], axis=0)
        f_s = jnp.concatenate([pad_f, f[:-s]], axis=0)
        comb = jnp.maximum(v, v_s) if is_max else v + v_s
        v = f * v + (1.0 - f) * comb
        f = jnp.maximum(f, f_s)
        s *= 2

    out_ref[...] = v
    cval[...] = v[-1:, :]
    cdst[...] = dst[-1:, :]


@functools.lru_cache(maxsize=None)
def _seg_scan(d, is_max):
    B = 1024
    grid = EP // B
    return pl.pallas_call(
        functools.partial(_scan_body, is_max),
        grid=(grid,),
        in_specs=[
            pl.BlockSpec((B, d), lambda b: (b, 0)),
            pl.BlockSpec((B, 1), lambda b: (b, 0)),
        ],
        out_specs=pl.BlockSpec((B, d), lambda b: (b, 0)),
        out_shape=jax.ShapeDtypeStruct((EP, d), jnp.float32),
        scratch_shapes=[
            pltpu.VMEM((1, d), jnp.float32),
            pltpu.VMEM((1, 1), jnp.float32),
        ],
    )


def _proj_body(x_ref, w_ref, o_ref):
    o_ref[...] = jax.lax.dot_general(
        x_ref[...], w_ref[...], (((1,), (0,)), ((), ())),
        preferred_element_type=jnp.float32)


@functools.lru_cache(maxsize=None)
def _proj(di, do):
    R = 1568
    return pl.pallas_call(
        _proj_body,
        grid=(NP // R,),
        in_specs=[
            pl.BlockSpec((R, di), lambda b: (b, 0)),
            pl.BlockSpec((di, do), lambda b: (0, 0)),
        ],
        out_specs=pl.BlockSpec((R, do), lambda b: (b, 0)),
        out_shape=jax.ShapeDtypeStruct((NP, do), jnp.float32),
    )


def _dense_body(mode, xs_ref, xb_ref, z0_ref, z1_ref, z2_ref, m3_ref,
                c1_ref, c2_ref, c3_ref, wr0_ref, wr1_ref, wr2_ref,
                wrs_ref, wrb_ref, bs_ref, bb_ref, hps_ref, hpb_ref,
                wres_s_ref, wres_b_ref, wlin_ref, blin_ref,
                os_ref, ob_ref):
    dot = lambda a, b: jax.lax.dot_general(
        a, b, (((1,), (0,)), ((), ())), preferred_element_type=jnp.float32)
    c1 = c1_ref[...]
    c2 = c2_ref[...]
    c3 = c3_ref[...]
    z1 = z1_ref[...] / jnp.maximum(c1, 1.0)
    z2 = z2_ref[...] / jnp.maximum(c2, 1.0)
    m3 = jnp.minimum(c3, 1.0) * m3_ref[...]

    ob = jax.nn.relu(dot(z2, wr2_ref[...]) + m3 +
                     dot(xb_ref[...], wrb_ref[...]) + bb_ref[...])
    if mode == "res_mm":
        ob = ob + dot(hpb_ref[...], wres_b_ref[...])
    elif mode == "res_id":
        ob = ob + hpb_ref[...]
    if mode == "final":
        ob = ob + hpb_ref[...]
        ob = dot(ob, wlin_ref[...]) + blin_ref[...]
    ob_ref[...] = ob

    if mode == "final":
        os_ref[...] = jnp.zeros_like(os_ref[...])
    else:
        os = jax.nn.relu(dot(z0_ref[...], wr0_ref[...]) + dot(z1, wr1_ref[...]) +
                         dot(xs_ref[...], wrs_ref[...]) + bs_ref[...])
        if mode == "res_mm":
            os = os + dot(hps_ref[...], wres_s_ref[...])
        elif mode == "res_id":
            os = os + hps_ref[...]
        os_ref[...] = os


@functools.lru_cache(maxsize=None)
def _dense(di, do, dp, dof, mode):
    R = 1568
    row = lambda d: pl.BlockSpec((R, d), lambda b: (b, 0))
    full = lambda a, c: pl.BlockSpec((a, c), lambda b: (0, 0))
    return pl.pallas_call(
        functools.partial(_dense_body, mode),
        grid=(NP // R,),
        in_specs=[
            row(di), row(di), row(di), row(di), row(di), row(do),
            row(1), row(1), row(1),
            full(di, do), full(di, do), full(di, do),
            full(di, do), full(di, do),
            full(1, do), full(1, do),
            row(dp), row(dp),
            full(dp, do), full(dp, do),
            full(do, dof), full(1, dof),
        ],
        out_specs=[row(do), row(dof)],
        out_shape=[jax.ShapeDtypeStruct((NP, do), jnp.float32),
                   jax.ShapeDtypeStruct((NP, dof), jnp.float32)],
    )


def _prep_edges(e):
    src = e[0].astype(jnp.int32)
    dst = e[1].astype(jnp.int32)
    perm = jnp.argsort(dst)
    src_s = src[perm]
    dst_s = dst[perm]
    pad_src = (jnp.arange(EP - EJ, dtype=jnp.int32) * 7919) % NSJ
    src_p = jnp.concatenate([src_s, pad_src])
    dst_p = jnp.concatenate([dst_s, jnp.full((EP - EJ,), NP + 5, jnp.int32)])
    off = jnp.searchsorted(dst_s, jnp.arange(NP + 1, dtype=jnp.int32)).astype(jnp.int32)
    last = jnp.maximum(off[1:] - 1, 0)
    cnt = (off[1:] - off[:-1]).astype(jnp.float32)[:, None]
    dst_f = dst_p.astype(jnp.float32)[:, None]
    return src_p, dst_f, last, cnt


def _segment_reduce(x_pad, src_p, dst_f, last, is_max):
    g = _gather_rows(x_pad, src_p, 25088, 448)
    s = _seg_scan(x_pad.shape[1], is_max)(g, dst_f)
    return _gather_rows(s, last, 1568, 392)


def _pad_rows(x, rows):
    return jnp.pad(x, ((0, rows - x.shape[0]), (0, 0)))


def _padw(w):
    return jnp.pad(w, ((0, 128 - w.shape[0]), (0, 128 - w.shape[1])))


def kernel(x_stroke, x_brep, edge_index_temp_previous, edge_index_intersects,
           edge_index_represented_by, edge_index_brepcoplanar,
           Wrel0, Wroot0, b0, Wrel1, Wroot1, b1, Wrel2, Wroot2, b2,
           Wrel3, Wroot3, b3, Wrel4, Wroot4, b4, Wres1, Wlin, blin):
    e0 = _prep_edges(edge_index_temp_previous)
    e1 = _prep_edges(edge_index_intersects)
    e2 = _prep_edges(edge_index_represented_by)
    e3 = _prep_edges(edge_index_brepcoplanar)

    xs = _pad_rows(jnp.pad(x_stroke, ((0, 0), (0, 122))), NP)
    xb = _pad_rows(jnp.pad(x_brep, ((0, 0), (0, 122))), NP)

    pw = lambda W: [_padw(W[i]) for i in range(4)]
    pb = lambda b: jnp.pad(b, ((0, 0), (0, 128 - b.shape[1])))
    zero = jnp.zeros((128, 128), jnp.float32)
    zrow = jnp.zeros((NP, 128), jnp.float32)

    layers = [
        (pw(Wrel0), pw(Wroot0), pb(b0), "plain"),
        (pw(Wrel1), pw(Wroot1), pb(b1), "res_mm"),
        (pw(Wrel2), pw(Wroot2), pb(b2), "res_id"),
        (pw(Wrel3), pw(Wroot3), pb(b3), "res_id"),
        (pw(Wrel4), pw(Wroot4), pb(b4), "final"),
    ]

    hs, hb = xs, xb
    for (Wr, Wq, bias, mode) in layers:
        z0 = _segment_reduce(hs, e0[0], e0[1], e0[2], False)
        z1 = _segment_reduce(hs, e1[0], e1[1], e1[2], False)
        z2 = _segment_reduce(hs, e2[0], e2[1], e2[2], False)
        y3 = _proj(128, 128)(hb, Wr[3])
        m3 = _segment_reduce(y3, e3[0], e3[1], e3[2], True)

        wrs = Wq[0] + Wq[1]
        wrb = Wq[2] + Wq[3]
        bs = (bias[0] + bias[1])[None, :]
        bb = (bias[2] + bias[3])[None, :]
        if mode == "res_mm":
            wres_s = _padw(Wres1[0])
            wres_b = _padw(Wres1[1])
        else:
            wres_s = zero
            wres_b = zero
        wlin = Wlin if mode == "final" else zero
        blin_ = blin[None, :] if mode == "final" else jnp.zeros((1, 128), jnp.float32)
        hps = zrow if mode == "plain" else hs
        hpb = zrow if mode == "plain" else hb

        os_, ob_ = _dense(128, 128, 128, 128, mode)(
            hs, hb, z0, z1, z2, m3, e1[3], e2[3], e3[3],
            Wr[0], Wr[1], Wr[2], wrs, wrb, bs, bb, hps, hpb,
            wres_s, wres_b, wlin, blin_)
        hs, hb = os_, ob_

    return hb[:NSJ]


# speedup vs baseline: 1.0706x; 1.0704x over previous
import functools
import jax
import jax.numpy as jnp
from jax import lax
from jax.experimental import pallas as pl
from jax.experimental.pallas import tpu as pltpu
from jax.experimental.pallas import tpu_sc as plsc

NSJ = 50000
EJ = 800000
NP = 50176
EP = 802816
NEG = -3.0e38


@functools.lru_cache(maxsize=None)
def _sc_gather(n_rows, d, b_per_w, ch):
    mesh = plsc.VectorSubcoreMesh(core_axis_name="c", subcore_axis_name="s")
    n_pair = b_per_w // (2 * ch)

    @functools.partial(
        pl.kernel, mesh=mesh,
        out_type=jax.ShapeDtypeStruct((n_rows * 0 + b_per_w * 32, d), jnp.float32),
        scratch_types=[
            pltpu.VMEM((ch,), jnp.int32),
            pltpu.VMEM((ch,), jnp.int32),
            pltpu.VMEM((ch, d), jnp.float32),
            pltpu.VMEM((ch, d), jnp.float32),
            pltpu.SemaphoreType.DMA,
            pltpu.SemaphoreType.DMA,
        ],
    )
    def k(table_hbm, idx_hbm, out_hbm, idx0_v, idx1_v, rows0_v, rows1_v, sem0, sem1):
        wid = lax.axis_index("s") * 2 + lax.axis_index("c")
        base = wid * b_per_w

        def body(p, carry):
            off = base + p * 2 * ch
            pltpu.sync_copy(idx_hbm.at[pl.ds(off, ch)], idx0_v)
            pltpu.sync_copy(idx_hbm.at[pl.ds(off + ch, ch)], idx1_v)
            c0 = pltpu.async_copy(table_hbm.at[idx0_v], rows0_v, sem0)
            c1 = pltpu.async_copy(table_hbm.at[idx1_v], rows1_v, sem1)
            c0.wait()
            pltpu.sync_copy(rows0_v, out_hbm.at[pl.ds(off, ch)])
            c1.wait()
            pltpu.sync_copy(rows1_v, out_hbm.at[pl.ds(off + ch, ch)])
            return carry

        lax.fori_loop(0, n_pair, body, 0)

    return k


def _gather_rows(table, idx, b_per_w, ch):
    return _sc_gather(table.shape[0], table.shape[1], b_per_w, ch)(table, idx)


def _scan_body(is_max, g_ref, d_ref, out_ref, cval, cdst):
    b = pl.program_id(0)
    ident = NEG if is_max else 0.0

    @pl.when(b == 0)
    def _():
        cval[...] = jnp.full_like(cval[...], ident)
        cdst[...] = jnp.full_like(cdst[...], -1.0)

    v = g_ref[...]
    dst = d_ref[...]
    B = v.shape[0]

    d_prev = jnp.concatenate([cdst[...], dst[:-1]], axis=0)
    f = jnp.minimum(jnp.abs(dst - d_prev), 1.0)

    row0 = jnp.concatenate(
        [jnp.ones((1, 1), jnp.float32), jnp.zeros((B - 1, 1), jnp.float32)], axis=0)
    cont0 = row0 * (1.0 - f)
    cv = jnp.broadcast_to(cval[...], v.shape)
    if is_max:
        v = jnp.maximum(v, cont0 * cv + (1.0 - cont0) * NEG)
    else:
        v = v + cont0 * cv

    s = 1
    while s < B:
        pad_v = jnp.full((s, v.shape[1]), ident, jnp.float32)
        pad_f = jnp.ones((s, 1), jnp.float32)
        v_s = jnp.concatenate([pad_v, v[:-s]], axis=0)
        f_s = jnp.concatenate([pad_f, f[:-s]], axis=0)
        comb = jnp.maximum(v, v_s) if is_max else v + v_s
        v = f * v + (1.0 - f) * comb
        f = jnp.maximum(f, f_s)
        s *= 2

    out_ref[...] = v
    cval[...] = v[-1:, :]
    cdst[...] = dst[-1:, :]


@functools.lru_cache(maxsize=None)
def _seg_scan(d, is_max):
    B = 1024
    grid = EP // B
    return pl.pallas_call(
        functools.partial(_scan_body, is_max),
        grid=(grid,),
        in_specs=[
            pl.BlockSpec((B, d), lambda b: (b, 0)),
            pl.BlockSpec((B, 1), lambda b: (b, 0)),
        ],
        out_specs=pl.BlockSpec((B, d), lambda b: (b, 0)),
        out_shape=jax.ShapeDtypeStruct((EP, d), jnp.float32),
        scratch_shapes=[
            pltpu.VMEM((1, d), jnp.float32),
            pltpu.VMEM((1, 1), jnp.float32),
        ],
    )


def _proj_body(x_ref, w_ref, o_ref):
    o_ref[...] = jax.lax.dot_general(
        x_ref[...], w_ref[...], (((1,), (0,)), ((), ())),
        preferred_element_type=jnp.float32)


@functools.lru_cache(maxsize=None)
def _proj(di, do):
    R = 1568
    return pl.pallas_call(
        _proj_body,
        grid=(NP // R,),
        in_specs=[
            pl.BlockSpec((R, di), lambda b: (b, 0)),
            pl.BlockSpec((di, do), lambda b: (0, 0)),
        ],
        out_specs=pl.BlockSpec((R, do), lambda b: (b, 0)),
        out_shape=jax.ShapeDtypeStruct((NP, do), jnp.float32),
    )


def _dense_body(mode, xs_ref, xb_ref, z0_ref, z1_ref, z2_ref, m3_ref,
                c1_ref, c2_ref, c3_ref, wr0_ref, wr1_ref, wr2_ref,
                wrs_ref, wrb_ref, bs_ref, bb_ref, hps_ref, hpb_ref,
                wres_s_ref, wres_b_ref, wlin_ref, blin_ref,
                os_ref, ob_ref):
    dot = lambda a, b: jax.lax.dot_general(
        a, b, (((1,), (0,)), ((), ())), preferred_element_type=jnp.float32)
    c1 = c1_ref[...]
    c2 = c2_ref[...]
    c3 = c3_ref[...]
    z1 = z1_ref[...] / jnp.maximum(c1, 1.0)
    z2 = z2_ref[...] / jnp.maximum(c2, 1.0)
    m3 = jnp.minimum(c3, 1.0) * m3_ref[...]

    ob = jax.nn.relu(dot(z2, wr2_ref[...]) + m3 +
                     dot(xb_ref[...], wrb_ref[...]) + bb_ref[...])
    if mode == "res_mm":
        ob = ob + dot(hpb_ref[...], wres_b_ref[...])
    elif mode == "res_id":
        ob = ob + hpb_ref[...]
    if mode == "final":
        ob = ob + hpb_ref[...]
        ob = dot(ob, wlin_ref[...]) + blin_ref[...]
    ob_ref[...] = ob

    if mode == "final":
        os_ref[...] = jnp.zeros_like(os_ref[...])
    else:
        os = jax.nn.relu(dot(z0_ref[...], wr0_ref[...]) + dot(z1, wr1_ref[...]) +
                         dot(xs_ref[...], wrs_ref[...]) + bs_ref[...])
        if mode == "res_mm":
            os = os + dot(hps_ref[...], wres_s_ref[...])
        elif mode == "res_id":
            os = os + hps_ref[...]
        os_ref[...] = os


@functools.lru_cache(maxsize=None)
def _dense(di, do, dp, dof, mode):
    R = 1568
    row = lambda d: pl.BlockSpec((R, d), lambda b: (b, 0))
    full = lambda a, c: pl.BlockSpec((a, c), lambda b: (0, 0))
    return pl.pallas_call(
        functools.partial(_dense_body, mode),
        grid=(NP // R,),
        in_specs=[
            row(di), row(di), row(di), row(di), row(di), row(do),
            row(1), row(1), row(1),
            full(di, do), full(di, do), full(di, do),
            full(di, do), full(di, do),
            full(1, do), full(1, do),
            row(dp), row(dp),
            full(dp, do), full(dp, do),
            full(do, dof), full(1, dof),
        ],
        out_specs=[row(do), row(dof)],
        out_shape=[jax.ShapeDtypeStruct((NP, do), jnp.float32),
                   jax.ShapeDtypeStruct((NP, dof), jnp.float32)],
    )


def _prep_edges(e):
    src = e[0].astype(jnp.int32)
    dst = e[1].astype(jnp.int32)
    perm = jnp.argsort(dst)
    src_s = src[perm]
    dst_s = dst[perm]
    pad_src = (jnp.arange(EP - EJ, dtype=jnp.int32) * 7919) % NSJ
    src_p = jnp.concatenate([src_s, pad_src])
    dst_p = jnp.concatenate([dst_s, jnp.full((EP - EJ,), NP + 5, jnp.int32)])
    off = jnp.searchsorted(dst_s, jnp.arange(NP + 1, dtype=jnp.int32)).astype(jnp.int32)
    last = jnp.maximum(off[1:] - 1, 0)
    cnt = (off[1:] - off[:-1]).astype(jnp.float32)[:, None]
    dst_f = dst_p.astype(jnp.float32)[:, None]
    return src_p, dst_f, last, cnt


def _segment_reduce(x_pad, src_p, dst_f, last, is_max):
    g = _gather_rows(x_pad, src_p, 25088, 448)
    s = _seg_scan(x_pad.shape[1], is_max)(g, dst_f)
    return _gather_rows(s, last, 1568, 392)


def _pad_rows(x, rows):
    return jnp.pad(x, ((0, rows - x.shape[0]), (0, 0)))


def _padw(w):
    return jnp.pad(w, ((0, 128 - w.shape[0]), (0, 128 - w.shape[1])))


def kernel(x_stroke, x_brep, edge_index_temp_previous, edge_index_intersects,
           edge_index_represented_by, edge_index_brepcoplanar,
           Wrel0, Wroot0, b0, Wrel1, Wroot1, b1, Wrel2, Wroot2, b2,
           Wrel3, Wroot3, b3, Wrel4, Wroot4, b4, Wres1, Wlin, blin):
    e0 = _prep_edges(edge_index_temp_previous)
    e1 = _prep_edges(edge_index_intersects)
    e2 = _prep_edges(edge_index_represented_by)
    e3 = _prep_edges(edge_index_brepcoplanar)

    xs = _pad_rows(jnp.pad(x_stroke, ((0, 0), (0, 122))), NP)
    xb = _pad_rows(jnp.pad(x_brep, ((0, 0), (0, 122))), NP)

    pw = lambda W: [_padw(W[i]) for i in range(4)]
    pb = lambda b: jnp.pad(b, ((0, 0), (0, 128 - b.shape[1])))
    zero = jnp.zeros((128, 128), jnp.float32)
    zrow = jnp.zeros((NP, 128), jnp.float32)

    layers = [
        (pw(Wrel0), pw(Wroot0), pb(b0), "plain"),
        (pw(Wrel1), pw(Wroot1), pb(b1), "res_mm"),
        (pw(Wrel2), pw(Wroot2), pb(b2), "res_id"),
        (pw(Wrel3), pw(Wroot3), pb(b3), "res_id"),
        (pw(Wrel4), pw(Wroot4), pb(b4), "final"),
    ]

    hs, hb = xs, xb
    for (Wr, Wq, bias, mode) in layers:
        if mode == "final":
            z0, z1 = zrow, zrow
        else:
            z0 = _segment_reduce(hs, e0[0], e0[1], e0[2], False)
            z1 = _segment_reduce(hs, e1[0], e1[1], e1[2], False)
        z2 = _segment_reduce(hs, e2[0], e2[1], e2[2], False)
        y3 = _proj(128, 128)(hb, Wr[3])
        m3 = _segment_reduce(y3, e3[0], e3[1], e3[2], True)

        wrs = Wq[0] + Wq[1]
        wrb = Wq[2] + Wq[3]
        bs = (bias[0] + bias[1])[None, :]
        bb = (bias[2] + bias[3])[None, :]
        if mode == "res_mm":
            wres_s = _padw(Wres1[0])
            wres_b = _padw(Wres1[1])
        else:
            wres_s = zero
            wres_b = zero
        wlin = Wlin if mode == "final" else zero
        blin_ = blin[None, :] if mode == "final" else jnp.zeros((1, 128), jnp.float32)
        hps = zrow if mode == "plain" else hs
        hpb = zrow if mode == "plain" else hb

        os_, ob_ = _dense(128, 128, 128, 128, mode)(
            hs, hb, z0, z1, z2, m3, e1[3], e2[3], e3[3],
            Wr[0], Wr[1], Wr[2], wrs, wrb, bs, bb, hps, hpb,
            wres_s, wres_b, wlin, blin_)
        hs, hb = os_, ob_

    return hb[:NSJ]
